# baseline (speedup 1.0000x reference)
import functools

import jax
import jax.numpy as jnp
from jax import lax
from jax.experimental import pallas as pl
from jax.experimental.pallas import tpu as pltpu
from jax.experimental.pallas import tpu_sc as plsc

NC = 2
NS = 16
L = 16
DH = 128
KCH = 80

_PREC = jax.lax.Precision.HIGHEST


def _make_rwse_kernel(npad, nch, n_rwse):
    rpt = npad // NS
    mesh = plsc.VectorSubcoreMesh(core_axis_name="c", subcore_axis_name="s", num_cores=NC, num_subcores=NS)

    @functools.partial(
        pl.kernel,
        out_type=(
            jax.ShapeDtypeStruct((n_rwse, npad), jnp.float32),
            jax.ShapeDtypeStruct((npad,), jnp.float32),
            jax.ShapeDtypeStruct((NC, npad), jnp.float32),
        ),
        mesh=mesh,
        compiler_params=pltpu.CompilerParams(needs_layout_passes=False),
        scratch_types=[
            pltpu.VMEM((nch, KCH), jnp.int32),
            pltpu.VMEM((nch, KCH), jnp.int32),
            pltpu.VMEM((npad,), jnp.float32),
            pltpu.VMEM((rpt,), jnp.float32),
            pltpu.VMEM((rpt,), jnp.float32),
            pltpu.VMEM((KCH,), jnp.float32),
            pltpu.VMEM((KCH,), jnp.float32),
            pltpu.VMEM((rpt,), jnp.float32),
            pltpu.SemaphoreType.DMA,
            pltpu.SemaphoreType.DMA,
            pltpu.VMEM_SHARED((npad,), jnp.float32),
        ],
    )
    def rwse_kernel(src_hbm, dst_hbm, cols_hbm, invdeg_hbm, curb_hbm,
                    src_v, dst_v, cur_v, invdeg_v, accr_v, val0, val1, z_v,
                    ss0, ss1, acc_sh):
        cid = lax.axis_index("c")
        sid = lax.axis_index("s")
        row0 = sid * rpt

        pltpu.sync_copy(src_hbm.at[sid], src_v)
        pltpu.sync_copy(dst_hbm.at[sid], dst_v)

        def fill_ones(i, _):
            val0[pl.ds(i * L, L)] = jnp.ones((L,), jnp.float32)
            val1[pl.ds(i * L, L)] = jnp.ones((L,), jnp.float32)
            return 0
        lax.fori_loop(0, KCH // L, fill_ones, 0)

        def fill_zero(i, _):
            z_v[pl.ds(i * L, L)] = jnp.zeros((L,), jnp.float32)
            return 0
        lax.fori_loop(0, rpt // L, fill_zero, 0)

        pltpu.sync_copy(z_v, acc_sh.at[pl.ds(row0, rpt)])
        plsc.subcore_barrier()

        def scat_issue(j, V, SS):
            pltpu.async_copy(V, acc_sh.at[dst_v.at[j]], SS, add=True)

        def scat_wait(j, V, SS):
            pltpu.make_async_copy(V, acc_sh.at[dst_v.at[j]], SS).wait()

        def gath(j, V):
            for i in range(KCH // L):
                idx = src_v[j, pl.ds(i * L, L)]
                V[pl.ds(i * L, L)] = plsc.load_gather(cur_v, [idx])

        def scatter_sweep(fill):
            if fill:
                gath(0, val0)
            scat_issue(0, val0, ss0)
            if fill:
                gath(1, val1)
            scat_issue(1, val1, ss1)

            def pairb(t, _):
                j = 2 * t + 2
                scat_wait(j, val0, ss0)
                if fill:
                    gath(j, val0)
                scat_issue(j, val0, ss0)
                scat_wait(j + 1, val1, ss1)
                if fill:
                    gath(j + 1, val1)
                scat_issue(j + 1, val1, ss1)
                return 0
            lax.fori_loop(0, (nch - 2) // 2, pairb, 0)
            scat_wait(0, val0, ss0)
            scat_wait(0, val1, ss1)

        scatter_sweep(fill=False)
        plsc.subcore_barrier()

        pltpu.sync_copy(acc_sh.at[pl.ds(row0, rpt)], accr_v)

        def deg_fix(i, _):
            d = accr_v[pl.ds(i * L, L)]
            d = jnp.where(d == 0.0, 1.0, d)
            accr_v[pl.ds(i * L, L)] = d
            invdeg_v[pl.ds(i * L, L)] = 1.0 / d
            return 0
        lax.fori_loop(0, rpt // L, deg_fix, 0)

        @pl.when(cid == 0)
        def _():
            pltpu.sync_copy(accr_v, cols_hbm.at[0, pl.ds(row0, rpt)])
            pltpu.sync_copy(invdeg_v, invdeg_hbm.at[pl.ds(row0, rpt)])
        pltpu.sync_copy(accr_v, curb_hbm.at[cid, pl.ds(row0, rpt)])
        pltpu.sync_copy(z_v, acc_sh.at[pl.ds(row0, rpt)])
        plsc.subcore_barrier()
        pltpu.sync_copy(curb_hbm.at[cid], cur_v)

        for k in range(1, n_rwse):
            scatter_sweep(fill=True)
            plsc.subcore_barrier()

            pltpu.sync_copy(acc_sh.at[pl.ds(row0, rpt)], accr_v)

            def upd(i, _):
                accr_v[pl.ds(i * L, L)] = (accr_v[pl.ds(i * L, L)]
                                           * invdeg_v[pl.ds(i * L, L)])
                return 0
            lax.fori_loop(0, rpt // L, upd, 0)

            @pl.when(cid == 0)
            def _():
                pltpu.sync_copy(accr_v, cols_hbm.at[k, pl.ds(row0, rpt)])
            if k < n_rwse - 1:
                pltpu.sync_copy(accr_v, curb_hbm.at[cid, pl.ds(row0, rpt)])
                pltpu.sync_copy(z_v, acc_sh.at[pl.ds(row0, rpt)])
                plsc.subcore_barrier()
                pltpu.sync_copy(curb_hbm.at[cid], cur_v)

    return rwse_kernel


def _make_agg_kernel(npad, nch, with_gate):
    rpt = npad // NS
    mesh = plsc.VectorSubcoreMesh(core_axis_name="c", subcore_axis_name="s", num_cores=NC, num_subcores=NS)
    scratch = [
        pltpu.VMEM((KCH,), jnp.int32),
        pltpu.VMEM((KCH,), jnp.int32),
        pltpu.VMEM((KCH,), jnp.int32),
        pltpu.VMEM((KCH,), jnp.int32),
        pltpu.VMEM((KCH,), jnp.int32),
        pltpu.VMEM((KCH,), jnp.int32),
        pltpu.VMEM((KCH, DH), jnp.float32),
        pltpu.VMEM((KCH, DH), jnp.float32),
        pltpu.VMEM((KCH, DH), jnp.float32),
        pltpu.VMEM((KCH, DH), jnp.float32),
        pltpu.SemaphoreType.DMA,
        pltpu.SemaphoreType.DMA,
        pltpu.SemaphoreType.DMA,
        pltpu.SemaphoreType.DMA,
        pltpu.SemaphoreType.DMA,
        pltpu.SemaphoreType.DMA,
        pltpu.VMEM_SHARED((npad, DH), jnp.float32),
    ]

    def body(*refs):
        if with_gate:
            (table_hbm, gate_hbm, src_hbm, dst_hbm, p0_hbm, p1_hbm,
             *rest) = refs
        else:
            (table_hbm, src_hbm, dst_hbm, p0_hbm, p1_hbm, *rest) = refs
            gate_hbm = None
        (src0, src1, dst0, dst1, dsc0, dsc1, rows0, rows1, gate0, gate1,
         sg0, sg1, ss0, ss1, sp0, sp1, acc_sh) = rest
        slot = [
            dict(src=src0, dst=dst0, dsc=dsc0, rows=rows0, gate=gate0,
                 sg=sg0, ss=ss0, sp=sp0),
            dict(src=src1, dst=dst1, dsc=dsc1, rows=rows1, gate=gate1,
                 sg=sg1, ss=ss1, sp=sp1),
        ]
        cid = lax.axis_index("c")
        sid = lax.axis_index("s")
        w = cid * NS + sid
        row0 = sid * rpt

        def pre_issue(jj, S):
            off = (w * nch + jj) * KCH
            pltpu.async_copy(src_hbm.at[pl.ds(off, KCH)], S["src"], S["sp"])
            pltpu.async_copy(dst_hbm.at[pl.ds(off, KCH)], S["dst"], S["sp"])
            if with_gate:
                pltpu.async_copy(gate_hbm.at[w, jj], S["gate"], S["sp"])

        def pre_wait(jj, S):
            off = (w * nch + jj) * KCH
            pltpu.make_async_copy(src_hbm.at[pl.ds(off, KCH)], S["src"],
                                  S["sp"]).wait()
            pltpu.make_async_copy(dst_hbm.at[pl.ds(off, KCH)], S["dst"],
                                  S["sp"]).wait()
            if with_gate:
                pltpu.make_async_copy(gate_hbm.at[w, jj], S["gate"],
                                      S["sp"]).wait()

        def compute_and_scatter(S):
            if with_gate:
                def mul(i, _):
                    S_rows = S["rows"]
                    S_gate = S["gate"]
                    for r in range(2):
                        for c in range(DH // L):
                            sl = pl.ds(c * L, L)
                            S_rows[2 * i + r, sl] = (S_rows[2 * i + r, sl]
                                                     * S_gate[2 * i + r, sl])
                    return 0
                lax.fori_loop(0, KCH // 2, mul, 0)
            for c in range(KCH // L):
                sl = pl.ds(c * L, L)
                S["dsc"][sl] = S["dst"][sl]
            pltpu.async_copy(S["rows"], acc_sh.at[S["dsc"]], S["ss"],
                             add=True)

        def scat_wait(S):
            pltpu.make_async_copy(S["rows"], acc_sh.at[S["dsc"]],
                                  S["ss"]).wait()

        def gather_issue(S):
            pltpu.async_copy(table_hbm.at[S["src"]], S["rows"], S["sg"])

        def gather_wait(S):
            pltpu.make_async_copy(table_hbm.at[S["src"]], S["rows"],
                                  S["sg"]).wait()

        def proc(jj, A, B, skip_scat_wait=False, skip_pre=False):
            pre_wait(jj, A)
            if not skip_scat_wait:
                scat_wait(A)
            gather_issue(A)
            gather_wait(B)
            compute_and_scatter(B)
            if not skip_pre:
                pre_issue(jj + 1, B)

        def zero_rows(i, _):
            for c in range(DH // L):
                rows0[i, pl.ds(c * L, L)] = jnp.zeros((L,), jnp.float32)
            return 0
        lax.fori_loop(0, KCH, zero_rows, 0)
        for t in range(rpt // KCH):
            pltpu.sync_copy(rows0, acc_sh.at[pl.ds(row0 + t * KCH, KCH)])
        plsc.subcore_barrier()

        pre_issue(0, slot[0])
        pre_wait(0, slot[0])
        gather_issue(slot[0])
        pre_issue(1, slot[1])
        proc(1, slot[1], slot[0], skip_scat_wait=True)

        def pair(t, _):
            jj = 2 * t + 2
            proc(jj, slot[0], slot[1])
            proc(jj + 1, slot[1], slot[0])
            return 0
        lax.fori_loop(0, (nch - 3) // 2, pair, 0)

        last = nch - 1
        lA, lB = slot[last % 2], slot[1 - last % 2]
        proc(last, lA, lB, skip_pre=True)
        gather_wait(lA)
        compute_and_scatter(lA)
        scat_wait(lB)
        scat_wait(lA)
        plsc.subcore_barrier()

        @pl.when(cid == 0)
        def _():
            pltpu.sync_copy(acc_sh.at[pl.ds(row0, rpt)],
                            p0_hbm.at[pl.ds(row0, rpt)])

        @pl.when(cid == 1)
        def _():
            pltpu.sync_copy(acc_sh.at[pl.ds(row0, rpt)],
                            p1_hbm.at[pl.ds(row0, rpt)])

    return functools.partial(
        pl.kernel,
        out_type=(jax.ShapeDtypeStruct((npad, DH), jnp.float32),
                  jax.ShapeDtypeStruct((npad, DH), jnp.float32)),
        mesh=mesh,
        scratch_types=scratch,
        compiler_params=pltpu.CompilerParams(needs_layout_passes=False),
    )(body)


def _init_tc(nf, rw, w0, b0):
    npad, d_in = nf.shape
    n_rwse = rw.shape[1]
    B = 640

    def body(nf_ref, rw_ref, w0_ref, b0_ref, h0_ref, out_ref):
        h0 = jnp.concatenate([nf_ref[...], rw_ref[...]], axis=1)
        h0_ref[...] = h0
        out_ref[...] = jnp.maximum(
            jnp.dot(h0, w0_ref[...], precision=_PREC,
                    preferred_element_type=jnp.float32) + b0_ref[...], 0.0)

    return pl.pallas_call(
        body,
        grid=(npad // B,),
        in_specs=[
            pl.BlockSpec((B, d_in), lambda i: (i, 0)),
            pl.BlockSpec((B, n_rwse), lambda i: (i, 0)),
            pl.BlockSpec((d_in + n_rwse, DH), lambda i: (0, 0)),
            pl.BlockSpec((1, DH), lambda i: (0, 0)),
        ],
        out_specs=[pl.BlockSpec((B, DH), lambda i: (i, 0))] * 2,
        out_shape=[jax.ShapeDtypeStruct((npad, DH), jnp.float32)] * 2,
    )(nf, rw, w0, b0)


def _edge_gate_z_tc(et, wp, bp_col, wg, bg):
    d_e, E = et.shape
    BE = 6400

    def body(et_ref, wp_ref, bp_ref, wg_ref, bg_ref, out_ref):
        e_t = et_ref[...]
        proj_t = jnp.maximum(
            lax.dot_general(wp_ref[...], e_t, (((0,), (0,)), ((), ())),
                            precision=_PREC,
                            preferred_element_type=jnp.float32)
            + bp_ref[...], 0.0)
        g_t = lax.dot_general(wg_ref[...], e_t, (((0,), (0,)), ((), ())),
                              precision=_PREC,
                              preferred_element_type=jnp.float32) + bg_ref[...]
        gate = 1.0 / (1.0 + jnp.exp(-g_t))
        out_ref[...] = proj_t * gate

    return pl.pallas_call(
        body,
        grid=(E // BE,),
        in_specs=[
            pl.BlockSpec((d_e, BE), lambda i: (0, i)),
            pl.BlockSpec((d_e, d_e), lambda i: (0, 0)),
            pl.BlockSpec((d_e, 1), lambda i: (0, 0)),
            pl.BlockSpec((d_e, 1), lambda i: (0, 0)),
            pl.BlockSpec((1, 1), lambda i: (0, 0)),
        ],
        out_specs=pl.BlockSpec((d_e, BE), lambda i: (0, i)),
        out_shape=jax.ShapeDtypeStruct((d_e, E), jnp.float32),
    )(et, wp, bp_col, wg, bg)


def _edge_gate_expand_tc(zt, weh):
    d_e, E = zt.shape
    BE = 6400

    def body(zt_ref, weh_ref, out_ref):
        out_ref[...] = lax.dot_general(
            zt_ref[...], weh_ref[...], (((0,), (0,)), ((), ())),
            precision=_PREC, preferred_element_type=jnp.float32)

    return pl.pallas_call(
        body,
        grid=(E // BE,),
        in_specs=[
            pl.BlockSpec((d_e, BE), lambda i: (0, i)),
            pl.BlockSpec((d_e, DH), lambda i: (0, 0)),
        ],
        out_specs=pl.BlockSpec((BE, DH), lambda i: (i, 0)),
        out_shape=jax.ShapeDtypeStruct((E, DH), jnp.float32),
    )(zt, weh)


def _step_tc(p0, p1, out, invd, wma, wmb, cb, bm):
    npad = out.shape[0]
    B = 640

    def body(p0_ref, p1_ref, out_ref, invd_ref, wma_ref, wmb_ref, cb_ref,
             bm_ref, new_ref):
        agg = (p0_ref[...] + p1_ref[...]) * invd_ref[...]
        m = jnp.maximum(agg + out_ref[...] + cb_ref[...], 0.0)
        new_ref[...] = (
            jnp.dot(m, wma_ref[...], precision=_PREC,
                    preferred_element_type=jnp.float32)
            + jnp.dot(out_ref[...], wmb_ref[...], precision=_PREC,
                      preferred_element_type=jnp.float32)
            + bm_ref[...])

    return pl.pallas_call(
        body,
        grid=(npad // B,),
        in_specs=[
            pl.BlockSpec((B, DH), lambda i: (i, 0)),
            pl.BlockSpec((B, DH), lambda i: (i, 0)),
            pl.BlockSpec((B, DH), lambda i: (i, 0)),
            pl.BlockSpec((B, DH), lambda i: (i, 0)),
            pl.BlockSpec((DH, DH), lambda i: (0, 0)),
            pl.BlockSpec((DH, DH), lambda i: (0, 0)),
            pl.BlockSpec((1, DH), lambda i: (0, 0)),
            pl.BlockSpec((1, DH), lambda i: (0, 0)),
        ],
        out_specs=pl.BlockSpec((B, DH), lambda i: (i, 0)),
        out_shape=jax.ShapeDtypeStruct((npad, DH), jnp.float32),
    )(p0, p1, out, invd, wma, wmb, cb, bm)


def _final_tc(p0, p1, out, invd, init, wsa, wsb, bs):
    npad = out.shape[0]
    B = 640

    def body(p0_ref, p1_ref, out_ref, invd_ref, init_ref, wsa_ref, wsb_ref,
             bs_ref, new_ref):
        group = (p0_ref[...] + p1_ref[...]) * invd_ref[...]
        new_ref[...] = (
            jnp.dot(out_ref[...], wsa_ref[...], precision=_PREC,
                    preferred_element_type=jnp.float32)
            + jnp.dot(group, wsb_ref[...], precision=_PREC,
                      preferred_element_type=jnp.float32)
            + bs_ref[...] + init_ref[...])

    return pl.pallas_call(
        body,
        grid=(npad // B,),
        in_specs=[
            pl.BlockSpec((B, DH), lambda i: (i, 0)),
            pl.BlockSpec((B, DH), lambda i: (i, 0)),
            pl.BlockSpec((B, DH), lambda i: (i, 0)),
            pl.BlockSpec((B, DH), lambda i: (i, 0)),
            pl.BlockSpec((B, DH), lambda i: (i, 0)),
            pl.BlockSpec((DH, DH), lambda i: (0, 0)),
            pl.BlockSpec((DH, DH), lambda i: (0, 0)),
            pl.BlockSpec((1, DH), lambda i: (0, 0)),
        ],
        out_specs=pl.BlockSpec((B, DH), lambda i: (i, 0)),
        out_shape=jax.ShapeDtypeStruct((npad, DH), jnp.float32),
    )(p0, p1, out, invd, init, wsa, wsb, bs)


def kernel(n_feat, edge_index, e_feat, Wp, bp, Wg, bg, W_eh, conv_bias,
           W0, b0, Wm, bm, Ws, bs):
    n, d_in = n_feat.shape
    E = edge_index.shape[1]
    n_rwse = W0.shape[0] - d_in
    npad = ((n + NS * L - 1) // (NS * L)) * (NS * L)

    src = edge_index[0]
    dst = edge_index[1]

    nch_r = E // (NS * KCH)
    nch_m = E // (NC * NS * KCH)
    src_r = src.reshape(NS, nch_r, KCH)
    dst_r = dst.reshape(NS, nch_r, KCH)


    zt = _edge_gate_z_tc(e_feat.T, Wp, bp.reshape(-1, 1), Wg,
                         bg.reshape(1, 1))
    gate = _edge_gate_expand_tc(zt, W_eh)
    gate_m = gate.reshape(NC * NS, nch_m, KCH, DH)

    cols, invdeg, _ = _make_rwse_kernel(npad, nch_r, n_rwse)(src_r, dst_r)
    rw = cols.T
    invd = jnp.broadcast_to(invdeg.reshape(npad, 1), (npad, DH))

    nf_pad = jnp.pad(n_feat, ((0, npad - n), (0, 0)))
    h0, out = _init_tc(nf_pad, rw, W0, b0.reshape(1, DH))

    agg_g = _make_agg_kernel(npad, nch_m, with_gate=True)
    wma, wmb = Wm[:DH], Wm[DH:]
    cb = conv_bias.reshape(1, DH)
    bmr = bm.reshape(1, DH)
    for _ in range(3):
        pa, pb = agg_g(out, gate_m, src, dst)
        out = _step_tc(pa, pb, out, invd, wma, wmb, cb, bmr)

    agg_p = _make_agg_kernel(npad, nch_m, with_gate=False)
    pa, pb = agg_p(out, src, dst)
    out = _final_tc(pa, pb, out, invd, h0, Ws[:DH], Ws[DH:],
                    bs.reshape(1, DH))
    return out[:n]


# speedup vs baseline: 1.0063x; 1.0063x over previous
import functools

import jax
import jax.numpy as jnp
from jax import lax
from jax.experimental import pallas as pl
from jax.experimental.pallas import tpu as pltpu
from jax.experimental.pallas import tpu_sc as plsc

NC = 2
NS = 16
L = 16
DH = 128
KCH = 80

_PREC = jax.lax.Precision.HIGHEST


def _make_rwse_kernel(npad, nch, n_rwse):
    rpt = npad // NS
    mesh = plsc.VectorSubcoreMesh(core_axis_name="c", subcore_axis_name="s", num_cores=NC, num_subcores=NS)

    @functools.partial(
        pl.kernel,
        out_type=(
            jax.ShapeDtypeStruct((n_rwse, npad), jnp.float32),
            jax.ShapeDtypeStruct((npad,), jnp.float32),
        ),
        mesh=mesh,
        compiler_params=pltpu.CompilerParams(needs_layout_passes=False),
        scratch_types=[
            pltpu.VMEM((nch, KCH), jnp.int32),
            pltpu.VMEM((nch, KCH), jnp.int32),
            pltpu.VMEM((npad,), jnp.float32),
            pltpu.VMEM((npad,), jnp.float32),
            pltpu.VMEM((npad,), jnp.float32),
            pltpu.VMEM((KCH,), jnp.float32),
            pltpu.VMEM((KCH,), jnp.float32),
            pltpu.VMEM((rpt,), jnp.float32),
            pltpu.SemaphoreType.DMA,
            pltpu.SemaphoreType.DMA,
            pltpu.VMEM_SHARED((npad,), jnp.float32),
        ],
    )
    def rwse_kernel(src_hbm, dst_hbm, cols_hbm, invdeg_hbm,
                    src_v, dst_v, cur_v, invdeg_v, accr_v, val0, val1, z_v,
                    ss0, ss1, acc_sh):
        cid = lax.axis_index("c")
        sid = lax.axis_index("s")
        row0 = sid * rpt

        pltpu.sync_copy(src_hbm.at[sid], src_v)
        pltpu.sync_copy(dst_hbm.at[sid], dst_v)

        def fill_ones(i, _):
            val0[pl.ds(i * L, L)] = jnp.ones((L,), jnp.float32)
            val1[pl.ds(i * L, L)] = jnp.ones((L,), jnp.float32)
            return 0
        lax.fori_loop(0, KCH // L, fill_ones, 0)

        def fill_zero(i, _):
            z_v[pl.ds(i * L, L)] = jnp.zeros((L,), jnp.float32)
            return 0
        lax.fori_loop(0, rpt // L, fill_zero, 0)

        pltpu.sync_copy(z_v, acc_sh.at[pl.ds(row0, rpt)])
        plsc.subcore_barrier()

        def scat_issue(j, V, SS):
            pltpu.async_copy(V, acc_sh.at[dst_v.at[j]], SS, add=True)

        def scat_wait(j, V, SS):
            pltpu.make_async_copy(V, acc_sh.at[dst_v.at[j]], SS).wait()

        def gath(j, V):
            for i in range(KCH // L):
                idx = src_v[j, pl.ds(i * L, L)]
                V[pl.ds(i * L, L)] = plsc.load_gather(cur_v, [idx])

        def scatter_sweep(fill):
            if fill:
                gath(0, val0)
            scat_issue(0, val0, ss0)
            if fill:
                gath(1, val1)
            scat_issue(1, val1, ss1)

            def pairb(t, _):
                j = 2 * t + 2
                scat_wait(j, val0, ss0)
                if fill:
                    gath(j, val0)
                scat_issue(j, val0, ss0)
                scat_wait(j + 1, val1, ss1)
                if fill:
                    gath(j + 1, val1)
                scat_issue(j + 1, val1, ss1)
                return 0
            lax.fori_loop(0, (nch - 2) // 2, pairb, 0)
            scat_wait(0, val0, ss0)
            scat_wait(0, val1, ss1)

        scatter_sweep(fill=False)
        plsc.subcore_barrier()

        pltpu.sync_copy(acc_sh, accr_v)

        def deg_fix(i, _):
            d = accr_v[pl.ds(i * L, L)]
            d = jnp.where(d == 0.0, 1.0, d)
            cur_v[pl.ds(i * L, L)] = d
            invdeg_v[pl.ds(i * L, L)] = 1.0 / d
            return 0
        lax.fori_loop(0, npad // L, deg_fix, 0)

        @pl.when(cid == 0)
        def _():
            pltpu.sync_copy(cur_v.at[pl.ds(row0, rpt)],
                            cols_hbm.at[0, pl.ds(row0, rpt)])
            pltpu.sync_copy(invdeg_v.at[pl.ds(row0, rpt)],
                            invdeg_hbm.at[pl.ds(row0, rpt)])
        plsc.subcore_barrier()

        for k in range(1, n_rwse):
            pltpu.sync_copy(z_v, acc_sh.at[pl.ds(row0, rpt)])
            plsc.subcore_barrier()
            scatter_sweep(fill=True)
            plsc.subcore_barrier()

            pltpu.sync_copy(acc_sh, accr_v)

            def upd(i, _):
                cur_v[pl.ds(i * L, L)] = (accr_v[pl.ds(i * L, L)]
                                          * invdeg_v[pl.ds(i * L, L)])
                return 0
            lax.fori_loop(0, npad // L, upd, 0)

            @pl.when(cid == 0)
            def _():
                pltpu.sync_copy(cur_v.at[pl.ds(row0, rpt)],
                                cols_hbm.at[k, pl.ds(row0, rpt)])
            plsc.subcore_barrier()

    return rwse_kernel


def _make_agg_kernel(npad, nch, with_gate):
    rpt = npad // NS
    mesh = plsc.VectorSubcoreMesh(core_axis_name="c", subcore_axis_name="s", num_cores=NC, num_subcores=NS)
    scratch = [
        pltpu.VMEM((KCH,), jnp.int32),
        pltpu.VMEM((KCH,), jnp.int32),
        pltpu.VMEM((KCH,), jnp.int32),
        pltpu.VMEM((KCH,), jnp.int32),
        pltpu.VMEM((KCH,), jnp.int32),
        pltpu.VMEM((KCH,), jnp.int32),
        pltpu.VMEM((KCH, DH), jnp.float32),
        pltpu.VMEM((KCH, DH), jnp.float32),
        pltpu.VMEM((KCH, DH), jnp.float32),
        pltpu.VMEM((KCH, DH), jnp.float32),
        pltpu.SemaphoreType.DMA,
        pltpu.SemaphoreType.DMA,
        pltpu.SemaphoreType.DMA,
        pltpu.SemaphoreType.DMA,
        pltpu.SemaphoreType.DMA,
        pltpu.SemaphoreType.DMA,
        pltpu.VMEM_SHARED((npad, DH), jnp.float32),
    ]

    def body(*refs):
        if with_gate:
            (table_hbm, gate_hbm, src_hbm, dst_hbm, p0_hbm, p1_hbm,
             *rest) = refs
        else:
            (table_hbm, src_hbm, dst_hbm, p0_hbm, p1_hbm, *rest) = refs
            gate_hbm = None
        (src0, src1, dst0, dst1, dsc0, dsc1, rows0, rows1, gate0, gate1,
         sg0, sg1, ss0, ss1, sp0, sp1, acc_sh) = rest
        slot = [
            dict(src=src0, dst=dst0, dsc=dsc0, rows=rows0, gate=gate0,
                 sg=sg0, ss=ss0, sp=sp0),
            dict(src=src1, dst=dst1, dsc=dsc1, rows=rows1, gate=gate1,
                 sg=sg1, ss=ss1, sp=sp1),
        ]
        cid = lax.axis_index("c")
        sid = lax.axis_index("s")
        w = cid * NS + sid
        row0 = sid * rpt

        def pre_issue(jj, S):
            off = (w * nch + jj) * KCH
            pltpu.async_copy(src_hbm.at[pl.ds(off, KCH)], S["src"], S["sp"])
            pltpu.async_copy(dst_hbm.at[pl.ds(off, KCH)], S["dst"], S["sp"])
            if with_gate:
                pltpu.async_copy(gate_hbm.at[w, jj], S["gate"], S["sp"])

        def pre_wait(jj, S):
            off = (w * nch + jj) * KCH
            pltpu.make_async_copy(src_hbm.at[pl.ds(off, KCH)], S["src"],
                                  S["sp"]).wait()
            pltpu.make_async_copy(dst_hbm.at[pl.ds(off, KCH)], S["dst"],
                                  S["sp"]).wait()
            if with_gate:
                pltpu.make_async_copy(gate_hbm.at[w, jj], S["gate"],
                                      S["sp"]).wait()

        def compute_and_scatter(S):
            if with_gate:
                def mul(i, _):
                    S_rows = S["rows"]
                    S_gate = S["gate"]
                    for r in range(2):
                        for c in range(DH // L):
                            sl = pl.ds(c * L, L)
                            S_rows[2 * i + r, sl] = (S_rows[2 * i + r, sl]
                                                     * S_gate[2 * i + r, sl])
                    return 0
                lax.fori_loop(0, KCH // 2, mul, 0)
            for c in range(KCH // L):
                sl = pl.ds(c * L, L)
                S["dsc"][sl] = S["dst"][sl]
            pltpu.async_copy(S["rows"], acc_sh.at[S["dsc"]], S["ss"],
                             add=True)

        def scat_wait(S):
            pltpu.make_async_copy(S["rows"], acc_sh.at[S["dsc"]],
                                  S["ss"]).wait()

        def gather_issue(S):
            pltpu.async_copy(table_hbm.at[S["src"]], S["rows"], S["sg"])

        def gather_wait(S):
            pltpu.make_async_copy(table_hbm.at[S["src"]], S["rows"],
                                  S["sg"]).wait()

        def proc(jj, A, B, skip_scat_wait=False, skip_pre=False):
            pre_wait(jj, A)
            if not skip_scat_wait:
                scat_wait(A)
            gather_issue(A)
            gather_wait(B)
            compute_and_scatter(B)
            if not skip_pre:
                pre_issue(jj + 1, B)

        def zero_rows(i, _):
            for c in range(DH // L):
                rows0[i, pl.ds(c * L, L)] = jnp.zeros((L,), jnp.float32)
            return 0
        lax.fori_loop(0, KCH, zero_rows, 0)
        for t in range(rpt // KCH):
            pltpu.sync_copy(rows0, acc_sh.at[pl.ds(row0 + t * KCH, KCH)])
        plsc.subcore_barrier()

        pre_issue(0, slot[0])
        pre_wait(0, slot[0])
        gather_issue(slot[0])
        pre_issue(1, slot[1])
        proc(1, slot[1], slot[0], skip_scat_wait=True)

        def pair(t, _):
            jj = 2 * t + 2
            proc(jj, slot[0], slot[1])
            proc(jj + 1, slot[1], slot[0])
            return 0
        lax.fori_loop(0, (nch - 3) // 2, pair, 0)

        last = nch - 1
        lA, lB = slot[last % 2], slot[1 - last % 2]
        proc(last, lA, lB, skip_pre=True)
        gather_wait(lA)
        compute_and_scatter(lA)
        scat_wait(lB)
        scat_wait(lA)
        plsc.subcore_barrier()

        @pl.when(cid == 0)
        def _():
            pltpu.sync_copy(acc_sh.at[pl.ds(row0, rpt)],
                            p0_hbm.at[pl.ds(row0, rpt)])

        @pl.when(cid == 1)
        def _():
            pltpu.sync_copy(acc_sh.at[pl.ds(row0, rpt)],
                            p1_hbm.at[pl.ds(row0, rpt)])

    return functools.partial(
        pl.kernel,
        out_type=(jax.ShapeDtypeStruct((npad, DH), jnp.float32),
                  jax.ShapeDtypeStruct((npad, DH), jnp.float32)),
        mesh=mesh,
        scratch_types=scratch,
        compiler_params=pltpu.CompilerParams(needs_layout_passes=False),
    )(body)


def _init_tc(nf, rw, w0, b0):
    npad, d_in = nf.shape
    n_rwse = rw.shape[1]
    B = 640

    def body(nf_ref, rw_ref, w0_ref, b0_ref, h0_ref, out_ref):
        h0 = jnp.concatenate([nf_ref[...], rw_ref[...]], axis=1)
        h0_ref[...] = h0
        out_ref[...] = jnp.maximum(
            jnp.dot(h0, w0_ref[...], precision=_PREC,
                    preferred_element_type=jnp.float32) + b0_ref[...], 0.0)

    return pl.pallas_call(
        body,
        grid=(npad // B,),
        in_specs=[
            pl.BlockSpec((B, d_in), lambda i: (i, 0)),
            pl.BlockSpec((B, n_rwse), lambda i: (i, 0)),
            pl.BlockSpec((d_in + n_rwse, DH), lambda i: (0, 0)),
            pl.BlockSpec((1, DH), lambda i: (0, 0)),
        ],
        out_specs=[pl.BlockSpec((B, DH), lambda i: (i, 0))] * 2,
        out_shape=[jax.ShapeDtypeStruct((npad, DH), jnp.float32)] * 2,
    )(nf, rw, w0, b0)


def _edge_gate_z_tc(et, wp, bp_col, wg, bg):
    d_e, E = et.shape
    BE = 6400

    def body(et_ref, wp_ref, bp_ref, wg_ref, bg_ref, out_ref):
        e_t = et_ref[...]
        proj_t = jnp.maximum(
            lax.dot_general(wp_ref[...], e_t, (((0,), (0,)), ((), ())),
                            precision=_PREC,
                            preferred_element_type=jnp.float32)
            + bp_ref[...], 0.0)
        g_t = lax.dot_general(wg_ref[...], e_t, (((0,), (0,)), ((), ())),
                              precision=_PREC,
                              preferred_element_type=jnp.float32) + bg_ref[...]
        gate = 1.0 / (1.0 + jnp.exp(-g_t))
        out_ref[...] = proj_t * gate

    return pl.pallas_call(
        body,
        grid=(E // BE,),
        in_specs=[
            pl.BlockSpec((d_e, BE), lambda i: (0, i)),
            pl.BlockSpec((d_e, d_e), lambda i: (0, 0)),
            pl.BlockSpec((d_e, 1), lambda i: (0, 0)),
            pl.BlockSpec((d_e, 1), lambda i: (0, 0)),
            pl.BlockSpec((1, 1), lambda i: (0, 0)),
        ],
        out_specs=pl.BlockSpec((d_e, BE), lambda i: (0, i)),
        out_shape=jax.ShapeDtypeStruct((d_e, E), jnp.float32),
    )(et, wp, bp_col, wg, bg)


def _edge_gate_expand_tc(zt, weh):
    d_e, E = zt.shape
    BE = 6400

    def body(zt_ref, weh_ref, out_ref):
        out_ref[...] = lax.dot_general(
            zt_ref[...], weh_ref[...], (((0,), (0,)), ((), ())),
            precision=_PREC, preferred_element_type=jnp.float32)

    return pl.pallas_call(
        body,
        grid=(E // BE,),
        in_specs=[
            pl.BlockSpec((d_e, BE), lambda i: (0, i)),
            pl.BlockSpec((d_e, DH), lambda i: (0, 0)),
        ],
        out_specs=pl.BlockSpec((BE, DH), lambda i: (i, 0)),
        out_shape=jax.ShapeDtypeStruct((E, DH), jnp.float32),
    )(zt, weh)


def _step_tc(p0, p1, out, invd, wma, wmb, cb, bm):
    npad = out.shape[0]
    B = 640

    def body(p0_ref, p1_ref, out_ref, invd_ref, wma_ref, wmb_ref, cb_ref,
             bm_ref, new_ref):
        agg = (p0_ref[...] + p1_ref[...]) * invd_ref[...]
        m = jnp.maximum(agg + out_ref[...] + cb_ref[...], 0.0)
        new_ref[...] = (
            jnp.dot(m, wma_ref[...], precision=_PREC,
                    preferred_element_type=jnp.float32)
            + jnp.dot(out_ref[...], wmb_ref[...], precision=_PREC,
                      preferred_element_type=jnp.float32)
            + bm_ref[...])

    return pl.pallas_call(
        body,
        grid=(npad // B,),
        in_specs=[
            pl.BlockSpec((B, DH), lambda i: (i, 0)),
            pl.BlockSpec((B, DH), lambda i: (i, 0)),
            pl.BlockSpec((B, DH), lambda i: (i, 0)),
            pl.BlockSpec((B, DH), lambda i: (i, 0)),
            pl.BlockSpec((DH, DH), lambda i: (0, 0)),
            pl.BlockSpec((DH, DH), lambda i: (0, 0)),
            pl.BlockSpec((1, DH), lambda i: (0, 0)),
            pl.BlockSpec((1, DH), lambda i: (0, 0)),
        ],
        out_specs=pl.BlockSpec((B, DH), lambda i: (i, 0)),
        out_shape=jax.ShapeDtypeStruct((npad, DH), jnp.float32),
    )(p0, p1, out, invd, wma, wmb, cb, bm)


def _final_tc(p0, p1, out, invd, init, wsa, wsb, bs):
    npad = out.shape[0]
    B = 640

    def body(p0_ref, p1_ref, out_ref, invd_ref, init_ref, wsa_ref, wsb_ref,
             bs_ref, new_ref):
        group = (p0_ref[...] + p1_ref[...]) * invd_ref[...]
        new_ref[...] = (
            jnp.dot(out_ref[...], wsa_ref[...], precision=_PREC,
                    preferred_element_type=jnp.float32)
            + jnp.dot(group, wsb_ref[...], precision=_PREC,
                      preferred_element_type=jnp.float32)
            + bs_ref[...] + init_ref[...])

    return pl.pallas_call(
        body,
        grid=(npad // B,),
        in_specs=[
            pl.BlockSpec((B, DH), lambda i: (i, 0)),
            pl.BlockSpec((B, DH), lambda i: (i, 0)),
            pl.BlockSpec((B, DH), lambda i: (i, 0)),
            pl.BlockSpec((B, DH), lambda i: (i, 0)),
            pl.BlockSpec((B, DH), lambda i: (i, 0)),
            pl.BlockSpec((DH, DH), lambda i: (0, 0)),
            pl.BlockSpec((DH, DH), lambda i: (0, 0)),
            pl.BlockSpec((1, DH), lambda i: (0, 0)),
        ],
        out_specs=pl.BlockSpec((B, DH), lambda i: (i, 0)),
        out_shape=jax.ShapeDtypeStruct((npad, DH), jnp.float32),
    )(p0, p1, out, invd, init, wsa, wsb, bs)


def kernel(n_feat, edge_index, e_feat, Wp, bp, Wg, bg, W_eh, conv_bias,
           W0, b0, Wm, bm, Ws, bs):
    n, d_in = n_feat.shape
    E = edge_index.shape[1]
    n_rwse = W0.shape[0] - d_in
    npad = ((n + NS * L - 1) // (NS * L)) * (NS * L)

    src = edge_index[0]
    dst = edge_index[1]

    nch_r = E // (NS * KCH)
    nch_m = E // (NC * NS * KCH)
    src_r = src.reshape(NS, nch_r, KCH)
    dst_r = dst.reshape(NS, nch_r, KCH)


    zt = _edge_gate_z_tc(e_feat.T, Wp, bp.reshape(-1, 1), Wg,
                         bg.reshape(1, 1))
    gate = _edge_gate_expand_tc(zt, W_eh)
    gate_m = gate.reshape(NC * NS, nch_m, KCH, DH)

    cols, invdeg = _make_rwse_kernel(npad, nch_r, n_rwse)(src_r, dst_r)
    rw = cols.T
    invd = jnp.broadcast_to(invdeg.reshape(npad, 1), (npad, DH))

    nf_pad = jnp.pad(n_feat, ((0, npad - n), (0, 0)))
    h0, out = _init_tc(nf_pad, rw, W0, b0.reshape(1, DH))

    agg_g = _make_agg_kernel(npad, nch_m, with_gate=True)
    wma, wmb = Wm[:DH], Wm[DH:]
    cb = conv_bias.reshape(1, DH)
    bmr = bm.reshape(1, DH)
    for _ in range(3):
        pa, pb = agg_g(out, gate_m, src, dst)
        out = _step_tc(pa, pb, out, invd, wma, wmb, cb, bmr)

    agg_p = _make_agg_kernel(npad, nch_m, with_gate=False)
    pa, pb = agg_p(out, src, dst)
    out = _final_tc(pa, pb, out, invd, h0, Ws[:DH], Ws[DH:],
                    bs.reshape(1, DH))
    return out[:n]
